# baseline (speedup 1.0000x reference)
import functools

import jax
import jax.numpy as jnp
from jax import lax
from jax.experimental import pallas as pl
from jax.experimental.pallas import tpu as pltpu
from jax.experimental.pallas import tpu_sc as plsc

B = 16384
S = 20
D = 16
L = 16
NC, NS = 2, 16
NW = NC * NS
BPW = B // NW
CB = 128
NCH = BPW // CB
P = CB * S
NIDX = P // 128
NG = CB // L


def _body(table, act1, gum, zt,
          score_o, cidx_o, clk_o, ztn_o, rew_o,
          act_v, rows_v, gum_v, zt_v, score_pk, cidx_v, clk_v,
          ztn_v, rew_v, sem):
    wid = lax.axis_index("s") * NC + lax.axis_index("c")
    lane = lax.iota(jnp.int32, L)
    rew_acc = jnp.zeros((L,), jnp.int32)

    for k in range(NCH):
        base_b = wid * BPW + k * CB
        base_p = base_b * S

        pltpu.sync_copy(act1.at[pl.ds(base_p, P)], act_v)
        pltpu.sync_copy(gum.at[pl.ds(base_p, P)], gum_v)
        pltpu.sync_copy(zt.at[pl.ds(base_b * D, CB * D)], zt_v)
        cps = [pltpu.async_copy(table.at[act_v.at[pl.ds(j * 128, 128)]],
                                rows_v.at[pl.ds(j * 128, 128)], sem)
               for j in range(NIDX)]
        for cp in cps:
            cp.wait()

        def group(g, rew):
            bvec = g * L + lane
            ztc = [plsc.load_gather(zt_v, [bvec * D + d]) for d in range(D)]

            def sbody(s, carry):
                bv, bi = carry
                rpos = bvec * S + s
                rbase = rpos * D
                acc = ztc[0] * plsc.load_gather(
                    rows_v, [rpos, jnp.zeros((L,), jnp.int32)])
                for d in range(1, D):
                    acc = acc + ztc[d] * plsc.load_gather(
                        rows_v, [rpos, jnp.full((L,), d, jnp.int32)])
                plsc.store_scatter(score_pk, [rpos], acc)
                comb = acc + plsc.load_gather(gum_v, [rpos])
                upd = comb > bv
                bv = jnp.where(upd, comb, bv)
                bi = jnp.where(upd, jnp.full((L,), s, jnp.int32), bi)
                return bv, bi

            bv0 = jnp.full((L,), -jnp.inf, jnp.float32)
            bi0 = jnp.zeros((L,), jnp.int32)
            _, bi = lax.fori_loop(0, S, sbody, (bv0, bi0))

            cpos = bvec * S + bi
            clicks = plsc.load_gather(act_v, [cpos])
            cidx_v[pl.ds(g * L, L)] = bi
            clk_v[pl.ds(g * L, L)] = clicks
            for d in range(D):
                r = plsc.load_gather(rows_v, [cpos, jnp.full((L,), d, jnp.int32)])
                plsc.store_scatter(ztn_v, [bvec * D + d], (ztc[d] + r) * 0.5)
            return rew + plsc.all_reduce_population_count(clicks > 1)

        rew_acc = lax.fori_loop(0, NG, group, rew_acc)

        pltpu.sync_copy(score_pk, score_o.at[pl.ds(base_p, P)])
        pltpu.sync_copy(cidx_v, cidx_o.at[pl.ds(base_b, CB)])
        pltpu.sync_copy(clk_v, clk_o.at[pl.ds(base_b, CB)])
        pltpu.sync_copy(ztn_v, ztn_o.at[pl.ds(base_b * D, CB * D)])

    rew_v[...] = rew_acc.astype(jnp.float32)
    pltpu.sync_copy(rew_v, rew_o.at[pl.ds(wid * L, L)])


_sc_call = pl.kernel(
    _body,
    out_type=(
        jax.ShapeDtypeStruct((B * S,), jnp.float32),
        jax.ShapeDtypeStruct((B,), jnp.int32),
        jax.ShapeDtypeStruct((B,), jnp.int32),
        jax.ShapeDtypeStruct((B * D,), jnp.float32),
        jax.ShapeDtypeStruct((NW * L,), jnp.float32),
    ),
    mesh=plsc.VectorSubcoreMesh(core_axis_name="c", subcore_axis_name="s",
                                num_cores=NC, num_subcores=NS),
    compiler_params=pltpu.CompilerParams(needs_layout_passes=False,
                                         use_tc_tiling_on_sc=False),
    scratch_types=(
        pltpu.VMEM((P,), jnp.int32),
        pltpu.VMEM((P, D), jnp.float32),
        pltpu.VMEM((P,), jnp.float32),
        pltpu.VMEM((CB * D,), jnp.float32),
        pltpu.VMEM((P,), jnp.float32),
        pltpu.VMEM((CB,), jnp.int32),
        pltpu.VMEM((CB,), jnp.int32),
        pltpu.VMEM((CB * D,), jnp.float32),
        pltpu.VMEM((L,), jnp.float32),
        pltpu.SemaphoreType.DMA,
    ),
)


@jax.jit
def kernel(action, zt, itemvec):
    act1 = action.reshape(-1)
    ztf = zt.reshape(-1)
    gum = jax.random.gumbel(jax.random.key(42), (B, S), jnp.float32).reshape(-1)
    score_f, cidx, clk, ztn, rew = _sc_call(itemvec, act1, gum, ztf)
    return (score_f.reshape(B, S), cidx, clk, ztn.reshape(B, 1, D),
            rew.reshape(NW, L)[:, 0].sum())
